# Initial kernel scaffold; baseline (speedup 1.0000x reference)
import jax
import jax.numpy as jnp
from jax.experimental import pallas as pl


def kernel(x, edge_index, edge_weight, theta1, wp1, gamma1, beta1, theta2, wp2, gamma2, beta2):
    raise NotImplementedError("write your pallas kernel here")


# speedup vs baseline: 2.0650x; 2.0650x over previous
import functools

import jax
import jax.numpy as jnp
from jax import lax
from jax.experimental import pallas as pl
from jax.experimental.pallas import tpu as pltpu
from jax.experimental.pallas import tpu_sc as plsc

N = 10000
E = 320000
F = 128
NC = 2
NS = 16
NW = NC * NS
FPW = F // NW
EPW = E // NW
CH = 2000
G16 = 16

_f32 = jnp.float32
_i32 = jnp.int32


def _mesh():
    return plsc.VectorSubcoreMesh(core_axis_name="c", subcore_axis_name="s")


_SC_PARAMS = pltpu.CompilerParams(needs_layout_passes=False)


def _wid():
    return lax.axis_index("s") * NC + lax.axis_index("c")


@functools.partial(
    pl.kernel,
    mesh=_mesh(),
    compiler_params=_SC_PARAMS,
    out_type=[
        jax.ShapeDtypeStruct((NW, N), _f32),
        jax.ShapeDtypeStruct((NW, N), _f32),
    ],
    scratch_types=[
        pltpu.VMEM((N,), _f32),
        pltpu.VMEM((N,), _f32),
        pltpu.VMEM((CH,), _i32),
        pltpu.VMEM((CH,), _i32),
        pltpu.VMEM((CH,), _f32),
    ],
)
def _deg_partials(src_hbm, dst_hbm, w_hbm, dsp_hbm, ddp_hbm, ds_v, dd_v, src_v, dst_v, w_v):
    wid = _wid()
    zero16 = jnp.zeros((G16,), _f32)

    def zb(i, carry):
        ds_v[pl.ds(i * G16, G16)] = zero16
        dd_v[pl.ds(i * G16, G16)] = zero16
        return carry

    lax.fori_loop(0, N // G16, zb, 0)

    e0 = wid * EPW

    def chunk(ci, carry):
        base = e0 + ci * CH
        pltpu.sync_copy(src_hbm.at[pl.ds(base, CH)], src_v)
        pltpu.sync_copy(dst_hbm.at[pl.ds(base, CH)], dst_v)
        pltpu.sync_copy(w_hbm.at[pl.ds(base, CH)], w_v)

        def grp(g, c2):
            s16 = src_v[pl.ds(g * G16, G16)]
            d16 = dst_v[pl.ds(g * G16, G16)]
            w16 = w_v[pl.ds(g * G16, G16)] + 1e-6
            plsc.addupdate_scatter(ds_v, [s16], w16)
            plsc.addupdate_scatter(dd_v, [d16], w16)
            return c2

        lax.fori_loop(0, CH // G16, grp, 0)
        return carry

    lax.fori_loop(0, EPW // CH, chunk, 0)
    pltpu.sync_copy(ds_v, dsp_hbm.at[wid])
    pltpu.sync_copy(dd_v, ddp_hbm.at[wid])


def _deg_reduce(dsp, ddp):
    def body(dsp_ref, ddp_ref, r1_ref, r2_ref):
        r1_ref[...] = lax.rsqrt(jnp.sum(dsp_ref[...], axis=0) + 1e-6)
        r2_ref[...] = lax.rsqrt(jnp.sum(ddp_ref[...], axis=0) + 1e-6)

    return pl.pallas_call(
        body,
        out_shape=[
            jax.ShapeDtypeStruct((N,), _f32),
            jax.ShapeDtypeStruct((N,), _f32),
        ],
    )(dsp, ddp)


@functools.partial(
    pl.kernel,
    mesh=_mesh(),
    compiler_params=_SC_PARAMS,
    out_type=[
        jax.ShapeDtypeStruct((E,), _f32),
        jax.ShapeDtypeStruct((E,), _i32),
    ],
    scratch_types=[
        pltpu.VMEM((N,), _f32),
        pltpu.VMEM((N,), _f32),
        pltpu.VMEM((CH,), _i32),
        pltpu.VMEM((CH,), _i32),
        pltpu.VMEM((CH,), _f32),
        pltpu.VMEM((CH,), _f32),
        pltpu.VMEM((CH,), _i32),
    ],
)
def _edge_norm(src_hbm, dst_hbm, w_hbm, r1_hbm, r2_hbm, wn_hbm, pk_hbm,
               r1_v, r2_v, src_v, dst_v, w_v, wn_v, pk_v):
    wid = _wid()
    pltpu.sync_copy(r1_hbm, r1_v)
    pltpu.sync_copy(r2_hbm, r2_v)
    e0 = wid * EPW

    def chunk(ci, carry):
        base = e0 + ci * CH
        pltpu.sync_copy(src_hbm.at[pl.ds(base, CH)], src_v)
        pltpu.sync_copy(dst_hbm.at[pl.ds(base, CH)], dst_v)
        pltpu.sync_copy(w_hbm.at[pl.ds(base, CH)], w_v)

        def grp(g, c2):
            sl = pl.ds(g * G16, G16)
            s16 = src_v[sl]
            d16 = dst_v[sl]
            g1 = plsc.load_gather(r1_v, [s16])
            g2 = plsc.load_gather(r2_v, [d16])
            wn_v[sl] = (w_v[sl] + 1e-6) * g1 * g2
            pk_v[sl] = (s16 << 14) | d16
            return c2

        lax.fori_loop(0, CH // G16, grp, 0)
        pltpu.sync_copy(wn_v, wn_hbm.at[pl.ds(base, CH)])
        pltpu.sync_copy(pk_v, pk_hbm.at[pl.ds(base, CH)])
        return carry

    lax.fori_loop(0, EPW // CH, chunk, 0)


def _make_cheb(K):
    nouts = K - 1

    @functools.partial(
        pl.kernel,
        mesh=_mesh(),
        compiler_params=_SC_PARAMS,
        out_type=jax.ShapeDtypeStruct((nouts, F, N), _f32),
        scratch_types=[
            pltpu.VMEM((FPW, N), _f32),
            pltpu.VMEM((FPW, N), _f32),
            pltpu.VMEM((FPW, N), _f32),
            pltpu.VMEM((CH,), _i32),
            pltpu.VMEM((CH,), _f32),
        ],
    )
    def cheb(hT_hbm, pk_hbm, wn_hbm, tx_hbm, s0, s1, s2, pk_v, wn_v):
        wid = _wid()
        f0 = wid * FPW
        cvecs = [jnp.full((G16,), c, _i32) for c in range(FPW)]
        zero16 = jnp.zeros((G16,), _f32)

        pltpu.sync_copy(hT_hbm.at[pl.ds(f0, FPW), :], s0)

        P, Q, C, X = None, s0, s1, s2
        for k in range(1, K):
            def zb(i, carry, C=C):
                for c in range(FPW):
                    C[c, pl.ds(i * G16, G16)] = zero16
                return carry

            lax.fori_loop(0, N // G16, zb, 0)

            def chunk(ci, carry, Q=Q, C=C):
                base = ci * CH
                pltpu.sync_copy(pk_hbm.at[pl.ds(base, CH)], pk_v)
                pltpu.sync_copy(wn_hbm.at[pl.ds(base, CH)], wn_v)

                def grp(g, c2):
                    sl = pl.ds(g * G16, G16)
                    p16 = pk_v[sl]
                    w16 = wn_v[sl]
                    s16 = lax.shift_right_logical(p16, 14)
                    d16 = p16 & 16383
                    for c in range(FPW):
                        gv = plsc.load_gather(Q, [cvecs[c], s16])
                        plsc.addupdate_scatter(C, [cvecs[c], d16], gv * w16)
                    return c2

                lax.fori_loop(0, CH // G16, grp, 0)
                return carry

            lax.fori_loop(0, E // CH, chunk, 0)

            def ew(i, carry, P=P, C=C, k=k):
                for c in range(FPW):
                    sl = (c, pl.ds(i * G16, G16))
                    if k == 1:
                        C[sl] = -C[sl]
                    else:
                        C[sl] = (-2.0) * C[sl] - P[sl]
                return carry

            lax.fori_loop(0, N // G16, ew, 0)

            pltpu.sync_copy(C, tx_hbm.at[k - 1, pl.ds(f0, FPW), :])
            P, Q, C = Q, C, (X if k == 1 else P)

    return cheb


_cheb_1 = _make_cheb(6)
_cheb_2 = _make_cheb(10)


def _make_mix(K, transpose_out, residual):
    BN = 2048

    def body(*refs):
        if residual:
            hT_ref, tx_ref, th_ref, wp_ref, g_ref, b_ref, x_ref, o_ref = refs
        else:
            hT_ref, tx_ref, th_ref, wp_ref, g_ref, b_ref, o_ref = refs
        wpv = wp_ref[...]
        acc = jnp.zeros((BN, F), _f32)
        for k in range(K):
            w2 = th_ref[k, :][:, None] * wpv
            src = hT_ref[...] if k == 0 else tx_ref[k - 1]
            acc = acc + lax.dot_general(
                src, w2, (((0,), (0,)), ((), ())),
                preferred_element_type=_f32)
        acc = jnp.maximum(acc, 0.0)
        mu = jnp.mean(acc, axis=1, keepdims=True)
        var = jnp.mean((acc - mu) ** 2, axis=1, keepdims=True)
        acc = (acc - mu) * lax.rsqrt(var + 1e-6) * g_ref[...][None, :] \
            + b_ref[...][None, :]
        if residual:
            acc = acc + x_ref[...]
        o_ref[...] = acc.T if transpose_out else acc

    in_specs = [
        pl.BlockSpec((F, BN), lambda i: (0, i)),
        pl.BlockSpec((K - 1, F, BN), lambda i: (0, 0, i)),
        pl.BlockSpec((K, F), lambda i: (0, 0)),
        pl.BlockSpec((F, F), lambda i: (0, 0)),
        pl.BlockSpec((F,), lambda i: (0,)),
        pl.BlockSpec((F,), lambda i: (0,)),
    ]
    if residual:
        in_specs.append(pl.BlockSpec((BN, F), lambda i: (i, 0)))
    if transpose_out:
        out_spec = pl.BlockSpec((F, BN), lambda i: (0, i))
        out_shape = jax.ShapeDtypeStruct((F, N), _f32)
    else:
        out_spec = pl.BlockSpec((BN, F), lambda i: (i, 0))
        out_shape = jax.ShapeDtypeStruct((N, F), _f32)

    return pl.pallas_call(
        body,
        grid=(pl.cdiv(N, BN),),
        in_specs=in_specs,
        out_specs=out_spec,
        out_shape=out_shape,
    )


_mix_1 = _make_mix(6, transpose_out=True, residual=False)
_mix_2 = _make_mix(10, transpose_out=False, residual=True)


def kernel(x, edge_index, edge_weight, theta1, wp1, gamma1, beta1,
           theta2, wp2, gamma2, beta2):
    xT = x.T
    th1 = theta1[:, :, 0]
    th2 = theta2[:, :, 0]

    src = edge_index[0]
    dst = edge_index[1]
    dsp, ddp = _deg_partials(src, dst, edge_weight)
    r1, r2 = _deg_reduce(dsp, ddp)
    wn, pk = _edge_norm(src, dst, edge_weight, r1, r2)

    tx1 = _cheb_1(xT, pk, wn)
    h1T = _mix_1(xT, tx1, th1, wp1, gamma1, beta1)
    tx2 = _cheb_2(h1T, pk, wn)
    out = _mix_2(h1T, tx2, th2, wp2, gamma2, beta2, x)
    return out
